# baseline (device time: 128120 ns/iter reference)
import jax
import jax.numpy as jnp
from jax import lax
from jax.experimental import pallas as pl
from jax.experimental.pallas import tpu as pltpu

N_Y = 4
K = 8
HALF = 1024
CR = HALF // K


def kernel(partial, resid, gamma):
    _, m, d = partial.shape
    partial_bf = partial.reshape(m, d).astype(jnp.bfloat16)
    resid_bf = resid.astype(jnp.bfloat16)
    gamma2d = gamma.reshape(1, d)

    def body(p_ref, r_ref, g_ref, out_ref,
             fwd_in, rev_in, fwd_out, rev_out, norm_bf, x_in, resid_half,
             resid_sem,
             fwd_in_sems, rev_in_sems, x_in_sems,
             fwd_out_sems, rev_out_sems, x_out_sems, exit_sem):
        my_x = lax.axis_index("x")
        my_y = lax.axis_index("y")
        my_z = lax.axis_index("z")
        f32 = jnp.float32

        def rows(c):
            return pl.ds(my_x * HALF + c * CR, CR)

        def p_chunk(c):
            return p_ref[rows(c), :]

        def send_fwd(src_ref, c, to_y):
            pltpu.make_async_remote_copy(
                src_ref=src_ref, dst_ref=fwd_in.at[c],
                send_sem=fwd_out_sems.at[c], recv_sem=fwd_in_sems.at[c],
                device_id=(my_x, to_y, my_z),
                device_id_type=pl.DeviceIdType.MESH,
            ).start()

        def send_rev(src_ref, c, to_y):
            pltpu.make_async_remote_copy(
                src_ref=src_ref, dst_ref=rev_in.at[c],
                send_sem=rev_out_sems.at[c], recv_sem=rev_in_sems.at[c],
                device_id=(my_x, to_y, my_z),
                device_id_type=pl.DeviceIdType.MESH,
            ).start()

        def wait_in(buf, sems, c):
            pltpu.make_async_remote_copy(
                src_ref=buf.at[c], dst_ref=buf.at[c],
                send_sem=sems.at[c], recv_sem=sems.at[c],
                device_id=(my_x, my_y, my_z),
                device_id_type=pl.DeviceIdType.MESH,
            ).wait_recv()

        def wait_sent(src_ref, buf, sems, c):
            pltpu.make_async_remote_copy(
                src_ref=src_ref, dst_ref=buf.at[c],
                send_sem=sems.at[c], recv_sem=sems.at[c],
                device_id=(my_x, my_y, my_z),
                device_id_type=pl.DeviceIdType.MESH,
            ).wait_send()

        def ln_store(c, total):
            yv = total + resid_half[pl.ds(c * CR, CR), :].astype(f32)
            ms = jnp.mean(yv * yv, axis=-1, keepdims=True)
            o = yv * lax.rsqrt(ms + 1e-6) * g_ref[...]
            out_ref[rows(c), :] = o
            norm_bf[c] = o.astype(jnp.bfloat16)
            pltpu.make_async_remote_copy(
                src_ref=norm_bf.at[c], dst_ref=x_in.at[c],
                send_sem=x_out_sems.at[c], recv_sem=x_in_sems.at[c],
                device_id=(1 - my_x, my_y, my_z),
                device_id_type=pl.DeviceIdType.MESH,
            ).start()

        def nbr_ys(r):
            return [ny for ny in (r - 1, r + 1) if 0 <= ny < N_Y]

        barrier_sem = pltpu.get_barrier_semaphore()

        def entry_barrier(r):
            def _():
                for ny in nbr_ys(r):
                    pl.semaphore_signal(
                        barrier_sem, inc=1, device_id=(my_x, ny, my_z),
                        device_id_type=pl.DeviceIdType.MESH)
                pl.semaphore_signal(
                    barrier_sem, inc=1, device_id=(1 - my_x, r, my_z),
                    device_id_type=pl.DeviceIdType.MESH)
                pl.semaphore_wait(barrier_sem, len(nbr_ys(r)) + 1)
            return _

        for r in range(N_Y):
            pl.when(my_y == r)(entry_barrier(r))

        resid_dma = pltpu.make_async_copy(
            r_ref.at[pl.ds(my_x * HALF, HALF), :],
            resid_half.at[...], resid_sem)
        resid_dma.start()

        def end_role(send, in_buf, in_sems, sent_buf, sent_sems):
            for c in range(K):
                send(p_ref.at[rows(c), :], c)
            resid_dma.wait()
            for c in range(K):
                wait_in(in_buf, in_sems, c)
                ln_store(c, p_chunk(c).astype(f32) + in_buf[c].astype(f32))
            for c in range(K):
                wait_sent(p_ref.at[rows(c), :], in_buf, sent_sems, c)

        def role0():
            end_role(lambda s, c: send_fwd(s, c, 1),
                     rev_in, rev_in_sems, fwd_in, fwd_out_sems)

        def role3():
            end_role(lambda s, c: send_rev(s, c, 2),
                     fwd_in, fwd_in_sems, rev_in, rev_out_sems)

        def ln_deferred(c):
            ln_store(c, fwd_in[c].astype(f32) + p_chunk(c).astype(f32)
                     + rev_in[c].astype(f32))

        def role1():
            for c in range(K):
                wait_in(fwd_in, fwd_in_sems, c)
                fwd_out[c] = fwd_in[c] + p_chunk(c)
                send_fwd(fwd_out.at[c], c, 2)
                wait_in(rev_in, rev_in_sems, c)
                rev_out[c] = rev_in[c] + p_chunk(c)
                send_rev(rev_out.at[c], c, 0)
                if c == 0:
                    resid_dma.wait()
                else:
                    ln_deferred(c - 1)
            ln_deferred(K - 1)
            for c in range(K):
                wait_sent(fwd_out.at[c], fwd_in, fwd_out_sems, c)
                wait_sent(rev_out.at[c], rev_in, rev_out_sems, c)

        def role2():
            for c in range(K):
                wait_in(rev_in, rev_in_sems, c)
                rev_out[c] = rev_in[c] + p_chunk(c)
                send_rev(rev_out.at[c], c, 1)
                wait_in(fwd_in, fwd_in_sems, c)
                fwd_out[c] = fwd_in[c] + p_chunk(c)
                send_fwd(fwd_out.at[c], c, 3)
                if c == 0:
                    resid_dma.wait()
                else:
                    ln_deferred(c - 1)
            ln_deferred(K - 1)
            for c in range(K):
                wait_sent(rev_out.at[c], rev_in, rev_out_sems, c)
                wait_sent(fwd_out.at[c], fwd_in, fwd_out_sems, c)

        pl.when(my_y == 0)(role0)
        pl.when(my_y == 1)(role1)
        pl.when(my_y == 2)(role2)
        pl.when(my_y == 3)(role3)

        for c in range(K):
            wait_in(x_in, x_in_sems, c)
            out_ref[pl.ds((1 - my_x) * HALF + c * CR, CR), :] = (
                x_in[c].astype(f32))
        for c in range(K):
            wait_sent(norm_bf.at[c], x_in, x_out_sems, c)

        def exit_barrier(r):
            def _():
                for ny in nbr_ys(r):
                    pl.semaphore_signal(
                        exit_sem, inc=1, device_id=(my_x, ny, my_z),
                        device_id_type=pl.DeviceIdType.MESH)
                pl.semaphore_signal(
                    exit_sem, inc=1, device_id=(1 - my_x, r, my_z),
                    device_id_type=pl.DeviceIdType.MESH)
                pl.semaphore_wait(exit_sem, len(nbr_ys(r)) + 1)
            return _

        for r in range(N_Y):
            pl.when(my_y == r)(exit_barrier(r))

    cdim = (K, CR, d)
    return pl.pallas_call(
        body,
        out_shape=jax.ShapeDtypeStruct((m, d), jnp.float32),
        in_specs=[
            pl.BlockSpec(memory_space=pltpu.VMEM),
            pl.BlockSpec(memory_space=pltpu.MemorySpace.HBM),
            pl.BlockSpec(memory_space=pltpu.VMEM),
        ],
        out_specs=pl.BlockSpec(memory_space=pltpu.VMEM),
        scratch_shapes=[
            pltpu.VMEM(cdim, jnp.bfloat16),
            pltpu.VMEM(cdim, jnp.bfloat16),
            pltpu.VMEM(cdim, jnp.bfloat16),
            pltpu.VMEM(cdim, jnp.bfloat16),
            pltpu.VMEM(cdim, jnp.bfloat16),
            pltpu.VMEM(cdim, jnp.bfloat16),
            pltpu.VMEM((HALF, d), jnp.bfloat16),
            pltpu.SemaphoreType.DMA,
            pltpu.SemaphoreType.DMA((K,)),
            pltpu.SemaphoreType.DMA((K,)),
            pltpu.SemaphoreType.DMA((K,)),
            pltpu.SemaphoreType.DMA((K,)),
            pltpu.SemaphoreType.DMA((K,)),
            pltpu.SemaphoreType.DMA((K,)),
            pltpu.SemaphoreType.REGULAR,
        ],
        compiler_params=pltpu.CompilerParams(
            collective_id=0,
            vmem_limit_bytes=60 * 1024 * 1024,
        ),
    )(partial_bf, resid_bf, gamma2d)
